# Initial kernel scaffold; baseline (speedup 1.0000x reference)
import jax
import jax.numpy as jnp
from jax.experimental import pallas as pl


def kernel(x, edge_index, Wl1, bl1, Wr1, br1, att1, bias1, Wl2, bl2, Wr2, br2, att2, bias2):
    raise NotImplementedError("write your pallas kernel here")


# speedup vs baseline: 30.7063x; 30.7063x over previous
import functools

import jax
import jax.numpy as jnp
from jax import lax
from jax.experimental import pallas as pl
from jax.experimental.pallas import tpu as pltpu
from jax.experimental.pallas import tpu_sc as plsc

N_NODES = 10000
N_EDGES = 320000
DIN = 128
DH = 16
H = 8
D1 = H * DH
DOUT = 64
DF = 64

NC = 2
NS = 16
NW = NC * NS
NP = 10240
ROWS_PER_TILE = NP // NS
EB = 80
EDGES_PER_W = N_EDGES // NW
NBATCH = EDGES_PER_W // EB

_MESH = plsc.VectorSubcoreMesh(core_axis_name="c", subcore_axis_name="s",
                               num_cores=NC, num_subcores=NS)


_GATHER_DNUMS = lax.GatherDimensionNumbers(
    offset_dims=(), collapsed_slice_dims=(0,), start_index_map=(0,))


def _lane_rotate(v, idx):
    return lax.gather(v, idx[:, None], _GATHER_DNUMS, (1,),
                      mode=lax.GatherScatterMode.PROMISE_IN_BOUNDS)


def _all_reduce_lanes(v):
    for sh in (8, 4, 2, 1):
        idx = (lax.iota(jnp.int32, 16) + sh) & 15
        v = v + _lane_rotate(v, idx)
    return v


def _edge_kernel_body(nheads, xl_hbm, xr_hbm, src_hbm, dst_hbm,
                      att_hbm, num_out, den_out,
                      rows_l, rows_r, wbuf, sidx, didx, att_v, num_s, den_s,
                      sem1, sem2):
    nblk = DF // 16
    cid = lax.axis_index("c")
    sid = lax.axis_index("s")
    wid = sid * NC + cid

    def zrow(i, _):
        for q in range(nblk):
            rows_l[i, pl.ds(16 * q, 16)] = jnp.zeros((16,), jnp.float32)
        wbuf[i, :] = jnp.zeros((16,), jnp.float32)
        return 0
    lax.fori_loop(0, EB, zrow, 0)

    def zcopy(k, _):
        off = sid * ROWS_PER_TILE + k * EB
        pltpu.sync_copy(rows_l, num_s.at[pl.ds(off, EB)])
        pltpu.sync_copy(wbuf, den_s.at[pl.ds(off, EB)])
        return 0
    lax.fori_loop(0, ROWS_PER_TILE // EB, zcopy, 0)
    plsc.subcore_barrier()

    pltpu.sync_copy(att_hbm, att_v)
    att_vecs = [att_v[q, :] for q in range(nblk)]
    lane = lax.iota(jnp.int32, 16)
    bph = nblk // nheads

    pltpu.sync_copy(src_hbm.at[wid], sidx)
    pltpu.sync_copy(dst_hbm.at[wid], didx)

    def batch_body(b, _):
        cp1 = pltpu.async_copy(xl_hbm.at[sidx.at[b]], rows_l, sem1)
        cp2 = pltpu.async_copy(xr_hbm.at[didx.at[b]], rows_r, sem2)
        cp1.wait()
        cp2.wait()

        def edge_body(e, _):
            wrow = jnp.zeros((16,), jnp.float32)
            for h in range(nheads):
                hacc = jnp.zeros((16,), jnp.float32)
                lvs = []
                for q in range(bph):
                    blk = h * bph + q
                    lv = rows_l[e, pl.ds(16 * blk, 16)]
                    rv = rows_r[e, pl.ds(16 * blk, 16)]
                    lvs.append(lv)
                    s = lv + rv
                    ls = jnp.maximum(s, 0.2 * s)
                    hacc = hacc + ls * att_vecs[blk]
                wsplat = jnp.exp(_all_reduce_lanes(hacc))
                wrow = wsplat if nheads == 1 else jnp.where(lane == h, wsplat, wrow)
                for q in range(bph):
                    blk = h * bph + q
                    rows_l[e, pl.ds(16 * blk, 16)] = lvs[q] * wsplat
            wbuf[e, :] = wrow
            return 0

        lax.fori_loop(0, EB, edge_body, 0)
        pltpu.sync_copy(rows_l, num_s.at[didx.at[b]], add=True)
        pltpu.sync_copy(wbuf, den_s.at[didx.at[b]], add=True)
        return 0

    lax.fori_loop(0, NBATCH, batch_body, 0)
    plsc.subcore_barrier()

    def ocopy(k, _):
        off = sid * ROWS_PER_TILE + k * EB
        pltpu.sync_copy(num_s.at[pl.ds(off, EB)],
                        num_out.at[cid].at[pl.ds(off, EB)])
        pltpu.sync_copy(den_s.at[pl.ds(off, EB)],
                        den_out.at[cid].at[pl.ds(off, EB)])
        return 0
    lax.fori_loop(0, ROWS_PER_TILE // EB, ocopy, 0)


def _make_edge_kernel(nheads):
    body = functools.partial(_edge_kernel_body, nheads)
    return pl.kernel(
        body,
        out_type=(
            jax.ShapeDtypeStruct((NC, NP, DF), jnp.float32),
            jax.ShapeDtypeStruct((NC, NP, 16), jnp.float32),
        ),
        mesh=_MESH,
        compiler_params=pltpu.CompilerParams(use_tc_tiling_on_sc=False),
        scratch_types=[
            pltpu.VMEM((EB, DF), jnp.float32),
            pltpu.VMEM((EB, DF), jnp.float32),
            pltpu.VMEM((EB, 16), jnp.float32),
            pltpu.VMEM((NBATCH, EB), jnp.int32),
            pltpu.VMEM((NBATCH, EB), jnp.int32),
            pltpu.VMEM((DF // 16, 16), jnp.float32),
            pltpu.VMEM_SHARED((NP, DF), jnp.float32),
            pltpu.VMEM_SHARED((NP, 16), jnp.float32),
            pltpu.SemaphoreType.DMA,
            pltpu.SemaphoreType.DMA,
        ],
    )


_edge_l1 = _make_edge_kernel(4)
_edge_l2 = _make_edge_kernel(1)


def _proj1_body(x_ref, wl_ref, bl_ref, wr_ref, br_ref,
                xla_ref, xlb_ref, xra_ref, xrb_ref):
    xb = x_ref[...]
    dn = (((1,), (1,)), ((), ()))
    xl = lax.dot_general(xb, wl_ref[...], dn,
                         preferred_element_type=jnp.float32) + bl_ref[...]
    xr = lax.dot_general(xb, wr_ref[...], dn,
                         preferred_element_type=jnp.float32) + br_ref[...]
    xla_ref[...] = xl[:, :DF]
    xlb_ref[...] = xl[:, DF:]
    xra_ref[...] = xr[:, :DF]
    xrb_ref[...] = xr[:, DF:]


def _fuse_body(na0_ref, na1_ref, nb0_ref, nb1_ref,
               da0_ref, da1_ref, db0_ref, db1_ref, bias_ref,
               wl_ref, bl_ref, wr_ref, br_ref, xl_ref, xr_ref):
    numa = na0_ref[0] + na1_ref[0]
    numb = nb0_ref[0] + nb1_ref[0]
    dena = da0_ref[0] + da1_ref[0]
    denb = db0_ref[0] + db1_ref[0]
    num = jnp.concatenate([numa, numb], axis=1)
    den8 = jnp.concatenate([dena[:, :4], denb[:, :4]], axis=1)
    colh = lax.broadcasted_iota(jnp.int32, (8, D1), 1) // DH
    rowh = lax.broadcasted_iota(jnp.int32, (8, D1), 0)
    expand = (colh == rowh).astype(jnp.float32)
    denr = lax.dot_general(den8, expand, (((1,), (0,)), ((), ())),
                           preferred_element_type=jnp.float32)
    h = num / (denr + 1e-16) + bias_ref[...]
    h = jnp.where(h > 0, h, jnp.exp(h) - 1.0)
    dn = (((1,), (1,)), ((), ()))
    xl_ref[...] = lax.dot_general(h, wl_ref[...], dn,
                                  preferred_element_type=jnp.float32) + bl_ref[...]
    xr_ref[...] = lax.dot_general(h, wr_ref[...], dn,
                                  preferred_element_type=jnp.float32) + br_ref[...]


def _final_body(numa_ref, numb_ref, dena_ref, denb_ref, bias_ref, out_ref):
    num = numa_ref[0] + numb_ref[0]
    den = dena_ref[0] + denb_ref[0]
    out_ref[...] = num / (den[:, 0:1] + 1e-16) + bias_ref[...]


def kernel(x, edge_index, Wl1, bl1, Wr1, br1, att1, bias1,
           Wl2, bl2, Wr2, br2, att2, bias2):
    src = edge_index[0].reshape(NW, NBATCH, EB)
    dst = edge_index[1].reshape(NW, NBATCH, EB)

    grid1 = N_NODES // 1000
    xla, xlb, xra, xrb = pl.pallas_call(
        _proj1_body,
        grid=(grid1,),
        in_specs=[
            pl.BlockSpec((1000, DIN), lambda i: (i, 0)),
            pl.BlockSpec((D1, DIN), lambda i: (0, 0)),
            pl.BlockSpec((1, D1), lambda i: (0, 0)),
            pl.BlockSpec((D1, DIN), lambda i: (0, 0)),
            pl.BlockSpec((1, D1), lambda i: (0, 0)),
        ],
        out_specs=[pl.BlockSpec((1000, DF), lambda i: (i, 0))] * 4,
        out_shape=[jax.ShapeDtypeStruct((N_NODES, DF), jnp.float32)] * 4,
    )(x, Wl1, bl1.reshape(1, D1), Wr1, br1.reshape(1, D1))

    numa, dena = _edge_l1(xla, xra, src, dst, att1[:4])
    numb, denb = _edge_l1(xlb, xrb, src, dst, att1[4:])

    grid2 = NP // 1024
    xl2, xr2 = pl.pallas_call(
        _fuse_body,
        grid=(grid2,),
        in_specs=[
            pl.BlockSpec((1, 1024, DF), lambda i: (0, i, 0)),
            pl.BlockSpec((1, 1024, DF), lambda i: (1, i, 0)),
            pl.BlockSpec((1, 1024, DF), lambda i: (0, i, 0)),
            pl.BlockSpec((1, 1024, DF), lambda i: (1, i, 0)),
            pl.BlockSpec((1, 1024, 16), lambda i: (0, i, 0)),
            pl.BlockSpec((1, 1024, 16), lambda i: (1, i, 0)),
            pl.BlockSpec((1, 1024, 16), lambda i: (0, i, 0)),
            pl.BlockSpec((1, 1024, 16), lambda i: (1, i, 0)),
            pl.BlockSpec((1, D1), lambda i: (0, 0)),
            pl.BlockSpec((DOUT, D1), lambda i: (0, 0)),
            pl.BlockSpec((1, DOUT), lambda i: (0, 0)),
            pl.BlockSpec((DOUT, D1), lambda i: (0, 0)),
            pl.BlockSpec((1, DOUT), lambda i: (0, 0)),
        ],
        out_specs=[
            pl.BlockSpec((1024, DOUT), lambda i: (i, 0)),
            pl.BlockSpec((1024, DOUT), lambda i: (i, 0)),
        ],
        out_shape=[
            jax.ShapeDtypeStruct((NP, DOUT), jnp.float32),
            jax.ShapeDtypeStruct((NP, DOUT), jnp.float32),
        ],
    )(numa, numa, numb, numb, dena, dena, denb, denb, bias1.reshape(1, D1),
      Wl2, bl2.reshape(1, DOUT), Wr2, br2.reshape(1, DOUT))

    num2, den2 = _edge_l2(xl2, xr2, src, dst, att2.reshape(DOUT // 16, 16))

    out = pl.pallas_call(
        _final_body,
        grid=(grid2,),
        in_specs=[
            pl.BlockSpec((1, 1024, DOUT), lambda i: (0, i, 0)),
            pl.BlockSpec((1, 1024, DOUT), lambda i: (1, i, 0)),
            pl.BlockSpec((1, 1024, 16), lambda i: (0, i, 0)),
            pl.BlockSpec((1, 1024, 16), lambda i: (1, i, 0)),
            pl.BlockSpec((1, DOUT), lambda i: (0, 0)),
        ],
        out_specs=pl.BlockSpec((1024, DOUT), lambda i: (i, 0)),
        out_shape=jax.ShapeDtypeStruct((NP, DOUT), jnp.float32),
    )(num2, num2, den2, den2, bias2.reshape(1, DOUT))

    return out[:N_NODES]
